# initial kernel scaffold (unmeasured)
import jax
import jax.numpy as jnp
from jax import lax
from jax.experimental import pallas as pl
from jax.experimental.pallas import tpu as pltpu

N_DEV = 16
B = 128
D = 128
BTOT = N_DEV * B


def kernel(x, Win0, Wout0, Win1, Wout1, Win2, Wout2):
    def body(x_ref, win0_ref, wout0_ref, win1_ref, wout1_ref, win2_ref,
             wout2_ref, out_ref, xf_ref, rs_ref, p_ref,
             send_sems, ag_sems, rs_sems):
        my = lax.axis_index("i")

        def chunk(ref, idx):
            return ref.at[pl.ds(idx * B, B), :]

        xf_ref[pl.ds(my * B, B), :] = x_ref[:, :]
        sends = []
        for d in range(1, N_DEV):
            peer = lax.rem(my + d, N_DEV)
            c = pltpu.make_async_remote_copy(
                src_ref=x_ref,
                dst_ref=chunk(xf_ref, my),
                send_sem=send_sems.at[d - 1],
                recv_sem=ag_sems.at[my],
                device_id=(peer,),
                device_id_type=pl.DeviceIdType.MESH,
            )
            c.start()
            sends.append(c)
        for d in range(1, N_DEV):
            src = lax.rem(my + d, N_DEV)
            pltpu.make_async_remote_copy(
                src_ref=x_ref,
                dst_ref=chunk(xf_ref, src),
                send_sem=send_sems.at[d - 1],
                recv_sem=ag_sems.at[src],
                device_id=(src,),
                device_id_type=pl.DeviceIdType.MESH,
            ).wait_recv()
        for c in sends:
            c.wait_send()

        def layer(win_ref, wout_ref):
            h = jnp.maximum(
                jnp.dot(xf_ref[:, :], win_ref[:, :],
                        preferred_element_type=jnp.float32),
                0.0,
            )
            p_ref[:, :] = jnp.dot(h, wout_ref[:, :],
                                  preferred_element_type=jnp.float32)

            rs_ref[pl.ds(my * B, B), :] = p_ref[pl.ds(my * B, B), :]
            sends = []
            for d in range(1, N_DEV):
                peer = lax.rem(my + d, N_DEV)
                c = pltpu.make_async_remote_copy(
                    src_ref=chunk(p_ref, peer),
                    dst_ref=chunk(rs_ref, my),
                    send_sem=send_sems.at[d - 1],
                    recv_sem=rs_sems.at[my],
                    device_id=(peer,),
                    device_id_type=pl.DeviceIdType.MESH,
                )
                c.start()
                sends.append(c)
            for d in range(1, N_DEV):
                src = lax.rem(my + d, N_DEV)
                pltpu.make_async_remote_copy(
                    src_ref=x_ref,
                    dst_ref=chunk(rs_ref, src),
                    send_sem=send_sems.at[d - 1],
                    recv_sem=rs_sems.at[src],
                    device_id=(src,),
                    device_id_type=pl.DeviceIdType.MESH,
                ).wait_recv()
            for c in sends:
                c.wait_send()

            red = rs_ref[0:B, :]
            for s in range(1, N_DEV):
                red = red + rs_ref[s * B:(s + 1) * B, :]
            xf_ref[pl.ds(my * B, B), :] = red

            sends = []
            for d in range(1, N_DEV):
                peer = lax.rem(my + d, N_DEV)
                c = pltpu.make_async_remote_copy(
                    src_ref=chunk(xf_ref, my),
                    dst_ref=chunk(xf_ref, my),
                    send_sem=send_sems.at[d - 1],
                    recv_sem=ag_sems.at[my],
                    device_id=(peer,),
                    device_id_type=pl.DeviceIdType.MESH,
                )
                c.start()
                sends.append(c)
            for d in range(1, N_DEV):
                src = lax.rem(my + d, N_DEV)
                pltpu.make_async_remote_copy(
                    src_ref=x_ref,
                    dst_ref=chunk(xf_ref, src),
                    send_sem=send_sems.at[d - 1],
                    recv_sem=ag_sems.at[src],
                    device_id=(src,),
                    device_id_type=pl.DeviceIdType.MESH,
                ).wait_recv()
            for c in sends:
                c.wait_send()

        layer(win0_ref, wout0_ref)
        layer(win1_ref, wout1_ref)
        layer(win2_ref, wout2_ref)

        out_ref[:, :] = xf_ref[:, :]

    return pl.pallas_call(
        body,
        out_shape=jax.ShapeDtypeStruct((BTOT, D), jnp.float32),
        in_specs=[pl.BlockSpec(memory_space=pltpu.VMEM)] * 7,
        out_specs=pl.BlockSpec(memory_space=pltpu.VMEM),
        scratch_shapes=[
            pltpu.VMEM((BTOT, D), jnp.float32),
            pltpu.VMEM((BTOT, D), jnp.float32),
            pltpu.VMEM((BTOT, D), jnp.float32),
            pltpu.SemaphoreType.DMA((N_DEV - 1,)),
            pltpu.SemaphoreType.DMA((N_DEV,)),
            pltpu.SemaphoreType.DMA((N_DEV,)),
        ],
        compiler_params=pltpu.CompilerParams(collective_id=0),
    )(x, Win0, Wout0, Win1, Wout1, Win2, Wout2)


# baseline (device time: 106047 ns/iter reference)
import jax
import jax.numpy as jnp
from jax import lax
from jax.experimental import pallas as pl
from jax.experimental.pallas import tpu as pltpu

N_DEV = 16
B = 128
D = 128
BTOT = N_DEV * B


def kernel(x, Win0, Wout0, Win1, Wout1, Win2, Wout2):
    def body(x_ref, win0_ref, wout0_ref, win1_ref, wout1_ref, win2_ref,
             wout2_ref, out_ref, xf_ref, rs_ref, p_ref,
             send_sems, ag_sems, rs_sems):
        my = lax.axis_index("i")

        def chunk(ref, idx):
            return ref.at[pl.ds(idx * B, B), :]

        xf_ref[pl.ds(my * B, B), :] = x_ref[:, :]
        sends = []
        for d in range(1, N_DEV):
            peer = lax.rem(my + d, N_DEV)
            c = pltpu.make_async_remote_copy(
                src_ref=x_ref,
                dst_ref=chunk(xf_ref, my),
                send_sem=send_sems.at[d - 1],
                recv_sem=ag_sems.at[my],
                device_id=(peer,),
                device_id_type=pl.DeviceIdType.MESH,
            )
            c.start()
            sends.append(c)
        for d in range(1, N_DEV):
            src = lax.rem(my + d, N_DEV)
            pltpu.make_async_remote_copy(
                src_ref=x_ref,
                dst_ref=chunk(xf_ref, src),
                send_sem=send_sems.at[d - 1],
                recv_sem=ag_sems.at[src],
                device_id=(src,),
                device_id_type=pl.DeviceIdType.MESH,
            ).wait_recv()
        for c in sends:
            c.wait_send()

        def layer(win_ref, wout_ref):
            h = jnp.maximum(
                jnp.dot(xf_ref[:, :], win_ref[:, :],
                        preferred_element_type=jnp.float32),
                0.0,
            )
            p_ref[:, :] = jnp.dot(h, wout_ref[:, :],
                                  preferred_element_type=jnp.float32)

            rs_ref[pl.ds(my * B, B), :] = p_ref[pl.ds(my * B, B), :]
            sends = []
            for d in range(1, N_DEV):
                peer = lax.rem(my + d, N_DEV)
                c = pltpu.make_async_remote_copy(
                    src_ref=chunk(p_ref, peer),
                    dst_ref=chunk(rs_ref, my),
                    send_sem=send_sems.at[d - 1],
                    recv_sem=rs_sems.at[my],
                    device_id=(peer,),
                    device_id_type=pl.DeviceIdType.MESH,
                )
                c.start()
                sends.append(c)
            for d in range(1, N_DEV):
                src = lax.rem(my + d, N_DEV)
                pltpu.make_async_remote_copy(
                    src_ref=x_ref,
                    dst_ref=chunk(rs_ref, src),
                    send_sem=send_sems.at[d - 1],
                    recv_sem=rs_sems.at[src],
                    device_id=(src,),
                    device_id_type=pl.DeviceIdType.MESH,
                ).wait_recv()
            for c in sends:
                c.wait_send()

            red = rs_ref[0:B, :]
            for s in range(1, N_DEV):
                red = red + rs_ref[s * B:(s + 1) * B, :]
            xf_ref[pl.ds(my * B, B), :] = red

            sends = []
            for d in range(1, N_DEV):
                peer = lax.rem(my + d, N_DEV)
                c = pltpu.make_async_remote_copy(
                    src_ref=chunk(xf_ref, my),
                    dst_ref=chunk(xf_ref, my),
                    send_sem=send_sems.at[d - 1],
                    recv_sem=ag_sems.at[my],
                    device_id=(peer,),
                    device_id_type=pl.DeviceIdType.MESH,
                )
                c.start()
                sends.append(c)
            for d in range(1, N_DEV):
                src = lax.rem(my + d, N_DEV)
                pltpu.make_async_remote_copy(
                    src_ref=x_ref,
                    dst_ref=chunk(xf_ref, src),
                    send_sem=send_sems.at[d - 1],
                    recv_sem=ag_sems.at[src],
                    device_id=(src,),
                    device_id_type=pl.DeviceIdType.MESH,
                ).wait_recv()
            for c in sends:
                c.wait_send()

        layer(win0_ref, wout0_ref)
        layer(win1_ref, wout1_ref)
        layer(win2_ref, wout2_ref)

        out_ref[:, :] = xf_ref[:, :]

    return pl.pallas_call(
        body,
        out_shape=jax.ShapeDtypeStruct((BTOT, D), jnp.float32),
        in_specs=[pl.BlockSpec(memory_space=pltpu.VMEM)] * 7,
        out_specs=pl.BlockSpec(memory_space=pltpu.VMEM),
        scratch_shapes=[
            pltpu.VMEM((BTOT, D), jnp.float32),
            pltpu.VMEM((BTOT, D), jnp.float32),
            pltpu.VMEM((BTOT, D), jnp.float32),
            pltpu.SemaphoreType.DMA((N_DEV - 1,)),
            pltpu.SemaphoreType.DMA((N_DEV,)),
            pltpu.SemaphoreType.DMA((N_DEV,)),
        ],
    )(x, Win0, Wout0, Win1, Wout1, Win2, Wout2)


# device time: 70380 ns/iter; 1.5068x vs baseline; 1.5068x over previous
import jax
import jax.numpy as jnp
from jax import lax
from jax.experimental import pallas as pl
from jax.experimental.pallas import tpu as pltpu

N_DEV = 16
B = 128
D = 128
BTOT = N_DEV * B


def kernel(x, Win0, Wout0, Win1, Wout1, Win2, Wout2):
    def body(x_ref, win0_ref, wout0_ref, win1_ref, wout1_ref, win2_ref,
             wout2_ref, out_ref, xg_ref, rs_ref, pb_ref, xb_ref,
             send_sems, ag_sems, rs_sems):
        my = lax.axis_index("i")

        def chunk(ref, idx):
            return ref.at[pl.ds(idx * B, B), :]

        def a2a_send(src_slice_fn, dst_ref, recv_sems):
            sends = []
            for d in range(1, N_DEV):
                peer = lax.rem(my + d, N_DEV)
                c = pltpu.make_async_remote_copy(
                    src_ref=src_slice_fn(peer),
                    dst_ref=chunk(dst_ref, my),
                    send_sem=send_sems.at[d - 1],
                    recv_sem=recv_sems.at[my],
                    device_id=(peer,),
                    device_id_type=pl.DeviceIdType.MESH,
                )
                c.start()
                sends.append(c)
            return sends

        def a2a_wait(dst_ref, recv_sems, sends):
            for d in range(1, N_DEV):
                src = lax.rem(my + d, N_DEV)
                pltpu.make_async_remote_copy(
                    src_ref=xb_ref,
                    dst_ref=chunk(dst_ref, src),
                    send_sem=send_sems.at[d - 1],
                    recv_sem=recv_sems.at[src],
                    device_id=(src,),
                    device_id_type=pl.DeviceIdType.MESH,
                ).wait_recv()
            for c in sends:
                c.wait_send()

        xb_ref[:, :] = x_ref[:, :].astype(jnp.bfloat16)
        xg_ref[pl.ds(my * B, B), :] = xb_ref[:, :]
        sends = a2a_send(lambda peer: xb_ref, xg_ref, ag_sems)
        a2a_wait(xg_ref, ag_sems, sends)

        def layer(win_ref, wout_ref):
            xf = xg_ref[:, :].astype(jnp.float32)
            h = jnp.maximum(
                jnp.dot(xf, win_ref[:, :], preferred_element_type=jnp.float32),
                0.0,
            )
            p = jnp.dot(h, wout_ref[:, :], preferred_element_type=jnp.float32)
            pb_ref[:, :] = p.astype(jnp.bfloat16)

            rs_ref[pl.ds(my * B, B), :] = pb_ref[pl.ds(my * B, B), :]
            sends = a2a_send(lambda peer: chunk(pb_ref, peer), rs_ref, rs_sems)
            a2a_wait(rs_ref, rs_sems, sends)

            red = rs_ref[0:B, :].astype(jnp.float32)
            for s in range(1, N_DEV):
                red = red + rs_ref[s * B:(s + 1) * B, :].astype(jnp.float32)

            xg_ref[pl.ds(my * B, B), :] = red.astype(jnp.bfloat16)
            sends = a2a_send(lambda peer: chunk(xg_ref, my), xg_ref, ag_sems)
            a2a_wait(xg_ref, ag_sems, sends)

        layer(win0_ref, wout0_ref)
        layer(win1_ref, wout1_ref)
        layer(win2_ref, wout2_ref)

        out_ref[:, :] = xg_ref[:, :].astype(jnp.float32)

    return pl.pallas_call(
        body,
        out_shape=jax.ShapeDtypeStruct((BTOT, D), jnp.float32),
        in_specs=[pl.BlockSpec(memory_space=pltpu.VMEM)] * 7,
        out_specs=pl.BlockSpec(memory_space=pltpu.VMEM),
        scratch_shapes=[
            pltpu.VMEM((BTOT, D), jnp.bfloat16),
            pltpu.VMEM((BTOT, D), jnp.bfloat16),
            pltpu.VMEM((BTOT, D), jnp.bfloat16),
            pltpu.VMEM((B, D), jnp.bfloat16),
            pltpu.SemaphoreType.DMA((N_DEV - 1,)),
            pltpu.SemaphoreType.DMA((N_DEV,)),
            pltpu.SemaphoreType.DMA((N_DEV,)),
        ],
    )(x, Win0, Wout0, Win1, Wout1, Win2, Wout2)


# device time: 68111 ns/iter; 1.5570x vs baseline; 1.0333x over previous
import jax
import jax.numpy as jnp
from jax import lax
from jax.experimental import pallas as pl
from jax.experimental.pallas import tpu as pltpu

N_DEV = 16
B = 128
D = 128
BTOT = N_DEV * B


def kernel(x, Win0, Wout0, Win1, Wout1, Win2, Wout2):
    def body(x_ref, win0_ref, wout0_ref, win1_ref, wout1_ref, win2_ref,
             wout2_ref, out_ref, xg_ref, rs_ref, pb_ref, xb_ref,
             rs_send_sems, ag_send_sems, ag_sems, rs_sems):
        my = lax.axis_index("i")

        def chunk(ref, idx):
            return ref.at[pl.ds(idx * B, B), :]

        def recv_wait(dst_ref, recv_sems, src_idx):
            pltpu.make_async_remote_copy(
                src_ref=xb_ref,
                dst_ref=chunk(dst_ref, src_idx),
                send_sem=ag_send_sems.at[0],
                recv_sem=recv_sems.at[src_idx],
                device_id=(src_idx,),
                device_id_type=pl.DeviceIdType.MESH,
            ).wait_recv()

        def ag_broadcast(src_slice_fn, dst_ref):
            sends = []
            for d in range(1, N_DEV):
                peer = lax.rem(my + d, N_DEV)
                c = pltpu.make_async_remote_copy(
                    src_ref=src_slice_fn(peer),
                    dst_ref=chunk(dst_ref, my),
                    send_sem=ag_send_sems.at[d - 1],
                    recv_sem=ag_sems.at[my],
                    device_id=(peer,),
                    device_id_type=pl.DeviceIdType.MESH,
                )
                c.start()
                sends.append(c)
            return sends

        xb_ref[:, :] = x_ref[:, :].astype(jnp.bfloat16)
        xg_ref[pl.ds(my * B, B), :] = xb_ref[:, :]
        pending_ag = ag_broadcast(lambda peer: xb_ref, xg_ref)

        def layer(win_ref, wout_ref, pending_ag):
            winb = win_ref[:, :].astype(jnp.bfloat16)
            woutb = wout_ref[:, :].astype(jnp.bfloat16)

            rs_sends = []
            acc_own = None
            for k in range(N_DEV):
                s = lax.rem(my + k, N_DEV)
                if k > 0:
                    recv_wait(xg_ref, ag_sems, s)
                xs = xg_ref[pl.ds(s * B, B), :]
                h = jnp.maximum(
                    jnp.dot(xs, winb, preferred_element_type=jnp.float32),
                    0.0,
                ).astype(jnp.bfloat16)
                ps = jnp.dot(h, woutb, preferred_element_type=jnp.float32)
                if k == 0:
                    acc_own = ps
                else:
                    pb_ref[pl.ds(s * B, B), :] = ps.astype(jnp.bfloat16)
                    c = pltpu.make_async_remote_copy(
                        src_ref=chunk(pb_ref, s),
                        dst_ref=chunk(rs_ref, my),
                        send_sem=rs_send_sems.at[k - 1],
                        recv_sem=rs_sems.at[my],
                        device_id=(s,),
                        device_id_type=pl.DeviceIdType.MESH,
                    )
                    c.start()
                    rs_sends.append(c)

            red = acc_own
            for d in range(1, N_DEV):
                src = lax.rem(my + d, N_DEV)
                recv_wait(rs_ref, rs_sems, src)
            for s in range(N_DEV):
                red = jnp.where(
                    s == my, red,
                    red + rs_ref[s * B:(s + 1) * B, :].astype(jnp.float32),
                )

            for c in pending_ag:
                c.wait_send()
            xg_ref[pl.ds(my * B, B), :] = red.astype(jnp.bfloat16)
            new_ag = ag_broadcast(lambda peer: chunk(xg_ref, my), xg_ref)
            for c in rs_sends:
                c.wait_send()
            return new_ag, red

        pending_ag, _ = layer(win0_ref, wout0_ref, pending_ag)
        pending_ag, _ = layer(win1_ref, wout1_ref, pending_ag)
        pending_ag, red = layer(win2_ref, wout2_ref, pending_ag)

        out_ref[pl.ds(my * B, B), :] = red
        for d in range(1, N_DEV):
            src = lax.rem(my + d, N_DEV)
            recv_wait(xg_ref, ag_sems, src)
            out_ref[pl.ds(src * B, B), :] = (
                xg_ref[pl.ds(src * B, B), :].astype(jnp.float32)
            )
        for c in pending_ag:
            c.wait_send()

    return pl.pallas_call(
        body,
        out_shape=jax.ShapeDtypeStruct((BTOT, D), jnp.float32),
        in_specs=[pl.BlockSpec(memory_space=pltpu.VMEM)] * 7,
        out_specs=pl.BlockSpec(memory_space=pltpu.VMEM),
        scratch_shapes=[
            pltpu.VMEM((BTOT, D), jnp.bfloat16),
            pltpu.VMEM((BTOT, D), jnp.bfloat16),
            pltpu.VMEM((BTOT, D), jnp.bfloat16),
            pltpu.VMEM((B, D), jnp.bfloat16),
            pltpu.SemaphoreType.DMA((N_DEV - 1,)),
            pltpu.SemaphoreType.DMA((N_DEV - 1,)),
            pltpu.SemaphoreType.DMA((N_DEV,)),
            pltpu.SemaphoreType.DMA((N_DEV,)),
        ],
    )(x, Win0, Wout0, Win1, Wout1, Win2, Wout2)
